# Initial kernel scaffold; baseline (speedup 1.0000x reference)
import jax
import jax.numpy as jnp
from jax.experimental import pallas as pl


def kernel(x, edge_index, edge_attr, u, node_batch, edge_batch, num_nodes, num_edges, edge_W, edge_b, node_W, node_b, global_W, global_b):
    raise NotImplementedError("write your pallas kernel here")


# speedup vs baseline: 3.3147x; 3.3147x over previous
import functools

import jax
import jax.numpy as jnp
from jax import lax
from jax.experimental import pallas as pl
from jax.experimental.pallas import tpu as pltpu
from jax.experimental.pallas import tpu_sc as plsc

N_NODES = 10000
N_EDGES = 320000
D_FEAT = 128
D_EDGE = 16
D_GLOBAL = 128

NC, NS = 2, 16
NW = NC * NS
EPT = N_EDGES // NW
CH = 128
CPT = EPT // CH
TAIL = EPT - CPT * CH
CHT = CPT + 1
NPT = N_NODES // NS

HIGHEST = lax.Precision.HIGHEST


def _sc_body(ei_hbm, ea_hbm, xs_hbm, xr_hbm,
             gath_hbm, sentp_hbm, recvp_hbm,
             idx_row, idx_col, ea_b, gx_b, gr_b, out_b, ea_t,
             xs_sh, xr_sh, sent_sh, recv_sh, sem_in, sem_out, sem_ea):
    c = lax.axis_index("c")
    s = lax.axis_index("s")
    wid = c * NS + s
    ebase = wid * EPT

    @pl.when(s == 2)
    def _():
        pltpu.sync_copy(xs_hbm, xs_sh)

    @pl.when(s == 3)
    def _():
        pltpu.sync_copy(xr_hbm, xr_sh)

    pltpu.sync_copy(ei_hbm.at[0, pl.ds(wid * CHT, CHT)], idx_row)
    pltpu.sync_copy(ei_hbm.at[1, pl.ds(wid * CHT, CHT)], idx_col)

    zero16 = jnp.zeros((D_EDGE,), jnp.float32)

    def zero_tail(i, _):
        ea_t[i] = zero16
        return ()

    lax.fori_loop(TAIL, CH, zero_tail, (), unroll=8)

    def zero_body(i, _):
        out_b[i % CH] = zero16
        return ()

    lax.fori_loop(0, CH, zero_body, (), unroll=8)
    npc = NPT // CH
    for t in range(npc):
        pltpu.sync_copy(out_b, sent_sh.at[pl.ds(s * NPT + t * CH, CH)])
        pltpu.sync_copy(out_b, recv_sh.at[pl.ds(s * NPT + t * CH, CH)])
    rem = NPT - npc * CH
    pltpu.sync_copy(out_b.at[pl.ds(0, rem)],
                    sent_sh.at[pl.ds(s * NPT + npc * CH, rem)])
    pltpu.sync_copy(out_b.at[pl.ds(0, rem)],
                    recv_sh.at[pl.ds(s * NPT + npc * CH, rem)])
    plsc.subcore_barrier()

    def chunk_body(k, _):
        eoff = ebase + k * CH
        ir = idx_row.at[k]
        ic = idx_col.at[k]
        h1 = pltpu.async_copy(ea_hbm.at[pl.ds(eoff, CH)], ea_b, sem_ea)
        h2 = pltpu.async_copy(xs_sh.at[ir], gx_b, sem_in)
        h3 = pltpu.async_copy(xr_sh.at[ic], gr_b, sem_in)
        h1.wait(); h2.wait(); h3.wait()

        def add_body(i, _):
            out_b[i] = gx_b[i] + gr_b[i]
            return ()

        lax.fori_loop(0, CH, add_body, (), unroll=8)

        h4 = pltpu.async_copy(out_b, gath_hbm.at[pl.ds(eoff, CH)], sem_out)
        pltpu.sync_copy(ea_b, sent_sh.at[ir], add=True)
        pltpu.sync_copy(ea_b, recv_sh.at[ic], add=True)
        h4.wait()
        return ()

    lax.fori_loop(0, CPT, chunk_body, ())

    toff = ebase + CPT * CH
    ir = idx_row.at[CPT]
    ic = idx_col.at[CPT]
    h1 = pltpu.async_copy(ea_hbm.at[pl.ds(toff, TAIL)], ea_t.at[pl.ds(0, TAIL)],
                          sem_ea)
    h2 = pltpu.async_copy(xs_sh.at[ir], gx_b, sem_in)
    h3 = pltpu.async_copy(xr_sh.at[ic], gr_b, sem_in)
    h1.wait(); h2.wait(); h3.wait()

    def tail_add(i, _):
        out_b[i] = gx_b[i] + gr_b[i]
        return ()

    lax.fori_loop(0, TAIL, tail_add, ())
    h4 = pltpu.async_copy(out_b.at[pl.ds(0, TAIL)],
                          gath_hbm.at[pl.ds(toff, TAIL)], sem_out)
    pltpu.sync_copy(ea_t, sent_sh.at[ir], add=True)
    pltpu.sync_copy(ea_t, recv_sh.at[ic], add=True)
    h4.wait()

    plsc.subcore_barrier()

    @pl.when(s == 0)
    def _():
        pltpu.sync_copy(sent_sh, sentp_hbm.at[pl.ds(c * N_NODES, N_NODES)])

    @pl.when(s == 1)
    def _():
        pltpu.sync_copy(recv_sh, recvp_hbm.at[pl.ds(c * N_NODES, N_NODES)])


_sc_kernel = functools.partial(
    pl.kernel,
    out_type=(
        jax.ShapeDtypeStruct((N_EDGES, D_EDGE), jnp.float32),
        jax.ShapeDtypeStruct((NC * N_NODES, D_EDGE), jnp.float32),
        jax.ShapeDtypeStruct((NC * N_NODES, D_EDGE), jnp.float32),
    ),
    mesh=plsc.VectorSubcoreMesh(core_axis_name="c", subcore_axis_name="s"),
    compiler_params=pltpu.CompilerParams(use_tc_tiling_on_sc=False),
    scratch_types=[
        pltpu.VMEM((CHT, CH), jnp.int32),
        pltpu.VMEM((CHT, CH), jnp.int32),
        pltpu.VMEM((CH, D_EDGE), jnp.float32),
        pltpu.VMEM((CH, D_EDGE), jnp.float32),
        pltpu.VMEM((CH, D_EDGE), jnp.float32),
        pltpu.VMEM((CH, D_EDGE), jnp.float32),
        pltpu.VMEM((CH, D_EDGE), jnp.float32),
        pltpu.VMEM_SHARED((N_NODES, D_EDGE), jnp.float32),
        pltpu.VMEM_SHARED((N_NODES, D_EDGE), jnp.float32),
        pltpu.VMEM_SHARED((N_NODES, D_EDGE), jnp.float32),
        pltpu.VMEM_SHARED((N_NODES, D_EDGE), jnp.float32),
        pltpu.SemaphoreType.DMA,
        pltpu.SemaphoreType.DMA,
        pltpu.SemaphoreType.DMA,
    ],
)(_sc_body)


def _proj_body(x_ref, ws_ref, wr_ref, xs_ref, xr_ref):
    xb = x_ref[...]
    xs_ref[...] = jnp.dot(xb, ws_ref[...], precision=HIGHEST)
    xr_ref[...] = jnp.dot(xb, wr_ref[...], precision=HIGHEST)


def _edge_body(ea_ref, gath_ref, we_ref, u_ref, weg_ref, eb_ref, out_ref):
    ce = jnp.dot(u_ref[...], weg_ref[...], precision=HIGHEST) + eb_ref[...]
    out_ref[...] = (jnp.dot(ea_ref[...], we_ref[...], precision=HIGHEST)
                    + gath_ref[...] + ce)


def _node_body(x_ref, sp_ref, rp_ref, wnx_ref, wns_ref, wnr_ref, wng_ref,
               u_ref, nb_ref, wgu_ref, wgx_ref, wge_ref, gb_ref,
               xn_ref, un_ref, accx_ref, acce_ref):
    i = pl.program_id(0)
    xb = x_ref[...]
    sent = sp_ref[0] + sp_ref[1]
    recv = rp_ref[0] + rp_ref[1]
    cn = jnp.dot(u_ref[...], wng_ref[...], precision=HIGHEST) + nb_ref[...]
    xn_ref[...] = (jnp.dot(xb, wnx_ref[...], precision=HIGHEST)
                   + jnp.dot(sent, wns_ref[...], precision=HIGHEST)
                   + jnp.dot(recv, wnr_ref[...], precision=HIGHEST)
                   + cn)

    @pl.when(i == 0)
    def _():
        accx_ref[...] = jnp.zeros_like(accx_ref)
        acce_ref[...] = jnp.zeros_like(acce_ref)

    accx_ref[...] += jnp.sum(xb, axis=0, keepdims=True)
    acce_ref[...] += jnp.sum(sent, axis=0, keepdims=True)

    @pl.when(i == pl.num_programs(0) - 1)
    def _():
        un_ref[...] = (jnp.dot(u_ref[...], wgu_ref[...], precision=HIGHEST)
                       + jnp.dot(accx_ref[...], wgx_ref[...], precision=HIGHEST)
                       + jnp.dot(acce_ref[...], wge_ref[...], precision=HIGHEST)
                       + gb_ref[...])


_NB = 2000
_EB = 3200


def kernel(x, edge_index, edge_attr, u, node_batch, edge_batch, num_nodes,
           num_edges, edge_W, edge_b, node_W, node_b, global_W, global_b):
    f32 = jnp.float32
    ei = edge_index.astype(jnp.int32).reshape(2, NW, EPT)
    ei = jnp.pad(ei, ((0, 0), (0, 0), (0, CHT * CH - EPT)))
    ei = ei.reshape(2, NW * CHT, CH)

    we_e = edge_W[:D_EDGE]
    we_s = edge_W[D_EDGE:D_EDGE + D_FEAT]
    we_r = edge_W[D_EDGE + D_FEAT:D_EDGE + 2 * D_FEAT]
    we_g = edge_W[D_EDGE + 2 * D_FEAT:]
    wn_x = node_W[:D_FEAT]
    wn_s = node_W[D_FEAT:D_FEAT + D_EDGE]
    wn_r = node_W[D_FEAT + D_EDGE:D_FEAT + 2 * D_EDGE]
    wn_g = node_W[D_FEAT + 2 * D_EDGE:]
    wg_u = global_W[:D_GLOBAL]
    wg_x = global_W[D_GLOBAL:D_GLOBAL + D_FEAT]
    wg_e = global_W[D_GLOBAL + D_FEAT:]

    full = lambda shape: pl.BlockSpec(shape, lambda i: tuple(0 for _ in shape))

    xs, xr = pl.pallas_call(
        _proj_body,
        grid=(N_NODES // _NB,),
        in_specs=[
            pl.BlockSpec((_NB, D_FEAT), lambda i: (i, 0)),
            full((D_FEAT, D_EDGE)),
            full((D_FEAT, D_EDGE)),
        ],
        out_specs=[
            pl.BlockSpec((_NB, D_EDGE), lambda i: (i, 0)),
            pl.BlockSpec((_NB, D_EDGE), lambda i: (i, 0)),
        ],
        out_shape=[
            jax.ShapeDtypeStruct((N_NODES, D_EDGE), f32),
            jax.ShapeDtypeStruct((N_NODES, D_EDGE), f32),
        ],
    )(x, we_s, we_r)

    gath, sentp, recvp = _sc_kernel(ei, edge_attr, xs, xr)

    edge_attr_new = pl.pallas_call(
        _edge_body,
        grid=(N_EDGES // _EB,),
        in_specs=[
            pl.BlockSpec((_EB, D_EDGE), lambda i: (i, 0)),
            pl.BlockSpec((_EB, D_EDGE), lambda i: (i, 0)),
            full((D_EDGE, D_EDGE)),
            full((1, D_GLOBAL)),
            full((D_GLOBAL, D_EDGE)),
            full((1, D_EDGE)),
        ],
        out_specs=pl.BlockSpec((_EB, D_EDGE), lambda i: (i, 0)),
        out_shape=jax.ShapeDtypeStruct((N_EDGES, D_EDGE), f32),
    )(edge_attr, gath, we_e, u, we_g, edge_b.reshape(1, D_EDGE))

    sp3 = sentp.reshape(NC, N_NODES, D_EDGE)
    rp3 = recvp.reshape(NC, N_NODES, D_EDGE)

    x_new, u_new = pl.pallas_call(
        _node_body,
        grid=(N_NODES // _NB,),
        in_specs=[
            pl.BlockSpec((_NB, D_FEAT), lambda i: (i, 0)),
            pl.BlockSpec((NC, _NB, D_EDGE), lambda i: (0, i, 0)),
            pl.BlockSpec((NC, _NB, D_EDGE), lambda i: (0, i, 0)),
            full((D_FEAT, D_FEAT)),
            full((D_EDGE, D_FEAT)),
            full((D_EDGE, D_FEAT)),
            full((D_FEAT, D_FEAT)),
            full((1, D_GLOBAL)),
            full((1, D_FEAT)),
            full((D_GLOBAL, D_GLOBAL)),
            full((D_FEAT, D_GLOBAL)),
            full((D_EDGE, D_GLOBAL)),
            full((1, D_GLOBAL)),
        ],
        out_specs=[
            pl.BlockSpec((_NB, D_FEAT), lambda i: (i, 0)),
            pl.BlockSpec((1, D_GLOBAL), lambda i: (0, 0)),
        ],
        out_shape=[
            jax.ShapeDtypeStruct((N_NODES, D_FEAT), f32),
            jax.ShapeDtypeStruct((1, D_GLOBAL), f32),
        ],
        scratch_shapes=[
            pltpu.VMEM((1, D_FEAT), f32),
            pltpu.VMEM((1, D_EDGE), f32),
        ],
    )(x, sp3, rp3, wn_x, wn_s, wn_r, wn_g, u, node_b.reshape(1, D_FEAT),
      wg_u, wg_x, wg_e, global_b.reshape(1, D_GLOBAL))

    return (x_new, edge_attr_new, u_new)


# speedup vs baseline: 3.4638x; 1.0450x over previous
import functools

import jax
import jax.numpy as jnp
from jax import lax
from jax.experimental import pallas as pl
from jax.experimental.pallas import tpu as pltpu
from jax.experimental.pallas import tpu_sc as plsc

N_NODES = 10000
N_EDGES = 320000
D_FEAT = 128
D_EDGE = 16
D_GLOBAL = 128

NC, NS = 2, 16
NW = NC * NS
EPT = N_EDGES // NW
CH = 128
CPT = EPT // CH
TAIL = EPT - CPT * CH
CHT = CPT + 1
SUPN = 8
NSUP = CPT // SUPN
NPT = N_NODES // NS

HIGHEST = lax.Precision.HIGHEST


def _sc_body(ei_hbm, ea_hbm, xs_hbm, xr_hbm,
             gath_hbm, sentp_hbm, recvp_hbm,
             idx_row, idx_col, ea_s, gx_s, gr_s, out_s, ea_t,
             xs_sh, xr_sh, sent_sh, recv_sh, sem_in, sem_out, sem_ea, sem_sc):
    c = lax.axis_index("c")
    s = lax.axis_index("s")
    wid = c * NS + s
    ebase = wid * EPT

    @pl.when(s == 2)
    def _():
        pltpu.sync_copy(xs_hbm, xs_sh)

    @pl.when(s == 3)
    def _():
        pltpu.sync_copy(xr_hbm, xr_sh)

    pltpu.sync_copy(ei_hbm.at[0, pl.ds(wid * CHT, CHT)], idx_row)
    pltpu.sync_copy(ei_hbm.at[1, pl.ds(wid * CHT, CHT)], idx_col)

    zero16 = jnp.zeros((D_EDGE,), jnp.float32)

    def zero_tail(i, _):
        ea_t[i] = zero16
        return ()

    lax.fori_loop(TAIL, CH, zero_tail, (), unroll=8)

    def zero_body(i, _):
        out_s[i % CH] = zero16
        return ()

    lax.fori_loop(0, CH, zero_body, (), unroll=8)
    npc = NPT // CH
    for t in range(npc):
        pltpu.sync_copy(out_s.at[pl.ds(0, CH)], sent_sh.at[pl.ds(s * NPT + t * CH, CH)])
        pltpu.sync_copy(out_s.at[pl.ds(0, CH)], recv_sh.at[pl.ds(s * NPT + t * CH, CH)])
    rem = NPT - npc * CH
    pltpu.sync_copy(out_s.at[pl.ds(0, rem)],
                    sent_sh.at[pl.ds(s * NPT + npc * CH, rem)])
    pltpu.sync_copy(out_s.at[pl.ds(0, rem)],
                    recv_sh.at[pl.ds(s * NPT + npc * CH, rem)])
    plsc.subcore_barrier()

    def super_chunks(kb, nch, ea_s, gx_s, gr_s, out_s):
        eoff = ebase + kb * CH
        h_ea = pltpu.async_copy(ea_hbm.at[pl.ds(eoff, nch * CH)],
                                ea_s.at[pl.ds(0, nch * CH)], sem_ea)
        hs = []
        for k in range(nch):
            ir = idx_row.at[kb + k]
            ic = idx_col.at[kb + k]
            hs.append(pltpu.async_copy(
                xs_sh.at[ir], gx_s.at[pl.ds(k * CH, CH)], sem_in))
            hs.append(pltpu.async_copy(
                xr_sh.at[ic], gr_s.at[pl.ds(k * CH, CH)], sem_in))
        h_ea.wait()
        for h in hs:
            h.wait()

        def add_body(i, _):
            out_s[i] = gx_s[i] + gr_s[i]
            return ()

        lax.fori_loop(0, nch * CH, add_body, (), unroll=8)

        h_out = pltpu.async_copy(out_s.at[pl.ds(0, nch * CH)],
                                 gath_hbm.at[pl.ds(eoff, nch * CH)], sem_out)
        scs = []
        for k in range(nch):
            ir = idx_row.at[kb + k]
            ic = idx_col.at[kb + k]
            seg = ea_s.at[pl.ds(k * CH, CH)]
            scs.append(pltpu.async_copy(seg, sent_sh.at[ir], sem_sc, add=True))
            scs.append(pltpu.async_copy(seg, recv_sh.at[ic], sem_sc, add=True))
        h_out.wait()
        for h in scs:
            h.wait()

    def super_body(t, _):
        super_chunks(t * SUPN, SUPN, ea_s, gx_s, gr_s, out_s)
        return ()

    lax.fori_loop(0, NSUP, super_body, ())
    super_chunks(NSUP * SUPN, CPT - NSUP * SUPN, ea_s, gx_s, gr_s, out_s)

    toff = ebase + CPT * CH
    ir = idx_row.at[CPT]
    ic = idx_col.at[CPT]
    h1 = pltpu.async_copy(ea_hbm.at[pl.ds(toff, TAIL)], ea_t.at[pl.ds(0, TAIL)],
                          sem_ea)
    h2 = pltpu.async_copy(xs_sh.at[ir], gx_s.at[pl.ds(0, CH)], sem_in)
    h3 = pltpu.async_copy(xr_sh.at[ic], gr_s.at[pl.ds(0, CH)], sem_in)
    h1.wait(); h2.wait(); h3.wait()

    def tail_add(i, _):
        out_s[i] = gx_s[i] + gr_s[i]
        return ()

    lax.fori_loop(0, TAIL, tail_add, ())
    h4 = pltpu.async_copy(out_s.at[pl.ds(0, TAIL)],
                          gath_hbm.at[pl.ds(toff, TAIL)], sem_out)
    pltpu.sync_copy(ea_t, sent_sh.at[ir], add=True)
    pltpu.sync_copy(ea_t, recv_sh.at[ic], add=True)
    h4.wait()

    plsc.subcore_barrier()

    @pl.when(s == 0)
    def _():
        pltpu.sync_copy(sent_sh, sentp_hbm.at[pl.ds(c * N_NODES, N_NODES)])

    @pl.when(s == 1)
    def _():
        pltpu.sync_copy(recv_sh, recvp_hbm.at[pl.ds(c * N_NODES, N_NODES)])


_sc_kernel = functools.partial(
    pl.kernel,
    out_type=(
        jax.ShapeDtypeStruct((N_EDGES, D_EDGE), jnp.float32),
        jax.ShapeDtypeStruct((NC * N_NODES, D_EDGE), jnp.float32),
        jax.ShapeDtypeStruct((NC * N_NODES, D_EDGE), jnp.float32),
    ),
    mesh=plsc.VectorSubcoreMesh(core_axis_name="c", subcore_axis_name="s"),
    compiler_params=pltpu.CompilerParams(use_tc_tiling_on_sc=False),
    scratch_types=[
        pltpu.VMEM((CHT, CH), jnp.int32),
        pltpu.VMEM((CHT, CH), jnp.int32),
        pltpu.VMEM((SUPN * CH, D_EDGE), jnp.float32),
        pltpu.VMEM((SUPN * CH, D_EDGE), jnp.float32),
        pltpu.VMEM((SUPN * CH, D_EDGE), jnp.float32),
        pltpu.VMEM((SUPN * CH, D_EDGE), jnp.float32),
        pltpu.VMEM((CH, D_EDGE), jnp.float32),
        pltpu.VMEM_SHARED((N_NODES, D_EDGE), jnp.float32),
        pltpu.VMEM_SHARED((N_NODES, D_EDGE), jnp.float32),
        pltpu.VMEM_SHARED((N_NODES, D_EDGE), jnp.float32),
        pltpu.VMEM_SHARED((N_NODES, D_EDGE), jnp.float32),
        pltpu.SemaphoreType.DMA,
        pltpu.SemaphoreType.DMA,
        pltpu.SemaphoreType.DMA,
        pltpu.SemaphoreType.DMA,
    ],
)(_sc_body)


def _proj_body(x_ref, ws_ref, wr_ref, xs_ref, xr_ref):
    xb = x_ref[...]
    xs_ref[...] = jnp.dot(xb, ws_ref[...], precision=HIGHEST)
    xr_ref[...] = jnp.dot(xb, wr_ref[...], precision=HIGHEST)


def _edge_body(ea_ref, gath_ref, we_ref, u_ref, weg_ref, eb_ref, out_ref):
    ce = jnp.dot(u_ref[...], weg_ref[...], precision=HIGHEST) + eb_ref[...]
    out_ref[...] = (jnp.dot(ea_ref[...], we_ref[...], precision=HIGHEST)
                    + gath_ref[...] + ce)


def _node_body(x_ref, sp_ref, rp_ref, wnx_ref, wns_ref, wnr_ref, wng_ref,
               u_ref, nb_ref, wgu_ref, wgx_ref, wge_ref, gb_ref,
               xn_ref, un_ref, accx_ref, acce_ref):
    i = pl.program_id(0)
    xb = x_ref[...]
    sent = sp_ref[0] + sp_ref[1]
    recv = rp_ref[0] + rp_ref[1]
    cn = jnp.dot(u_ref[...], wng_ref[...], precision=HIGHEST) + nb_ref[...]
    xn_ref[...] = (jnp.dot(xb, wnx_ref[...], precision=HIGHEST)
                   + jnp.dot(sent, wns_ref[...], precision=HIGHEST)
                   + jnp.dot(recv, wnr_ref[...], precision=HIGHEST)
                   + cn)

    @pl.when(i == 0)
    def _():
        accx_ref[...] = jnp.zeros_like(accx_ref)
        acce_ref[...] = jnp.zeros_like(acce_ref)

    accx_ref[...] += jnp.sum(xb, axis=0, keepdims=True)
    acce_ref[...] += jnp.sum(sent, axis=0, keepdims=True)

    @pl.when(i == pl.num_programs(0) - 1)
    def _():
        un_ref[...] = (jnp.dot(u_ref[...], wgu_ref[...], precision=HIGHEST)
                       + jnp.dot(accx_ref[...], wgx_ref[...], precision=HIGHEST)
                       + jnp.dot(acce_ref[...], wge_ref[...], precision=HIGHEST)
                       + gb_ref[...])


_NB = 2000
_EB = 3200


def kernel(x, edge_index, edge_attr, u, node_batch, edge_batch, num_nodes,
           num_edges, edge_W, edge_b, node_W, node_b, global_W, global_b):
    f32 = jnp.float32
    ei = edge_index.astype(jnp.int32).reshape(2, NW, EPT)
    ei = jnp.pad(ei, ((0, 0), (0, 0), (0, CHT * CH - EPT)))
    ei = ei.reshape(2, NW * CHT, CH)

    we_e = edge_W[:D_EDGE]
    we_s = edge_W[D_EDGE:D_EDGE + D_FEAT]
    we_r = edge_W[D_EDGE + D_FEAT:D_EDGE + 2 * D_FEAT]
    we_g = edge_W[D_EDGE + 2 * D_FEAT:]
    wn_x = node_W[:D_FEAT]
    wn_s = node_W[D_FEAT:D_FEAT + D_EDGE]
    wn_r = node_W[D_FEAT + D_EDGE:D_FEAT + 2 * D_EDGE]
    wn_g = node_W[D_FEAT + 2 * D_EDGE:]
    wg_u = global_W[:D_GLOBAL]
    wg_x = global_W[D_GLOBAL:D_GLOBAL + D_FEAT]
    wg_e = global_W[D_GLOBAL + D_FEAT:]

    full = lambda shape: pl.BlockSpec(shape, lambda i: tuple(0 for _ in shape))

    xs, xr = pl.pallas_call(
        _proj_body,
        grid=(N_NODES // _NB,),
        in_specs=[
            pl.BlockSpec((_NB, D_FEAT), lambda i: (i, 0)),
            full((D_FEAT, D_EDGE)),
            full((D_FEAT, D_EDGE)),
        ],
        out_specs=[
            pl.BlockSpec((_NB, D_EDGE), lambda i: (i, 0)),
            pl.BlockSpec((_NB, D_EDGE), lambda i: (i, 0)),
        ],
        out_shape=[
            jax.ShapeDtypeStruct((N_NODES, D_EDGE), f32),
            jax.ShapeDtypeStruct((N_NODES, D_EDGE), f32),
        ],
    )(x, we_s, we_r)

    gath, sentp, recvp = _sc_kernel(ei, edge_attr, xs, xr)

    edge_attr_new = pl.pallas_call(
        _edge_body,
        grid=(N_EDGES // _EB,),
        in_specs=[
            pl.BlockSpec((_EB, D_EDGE), lambda i: (i, 0)),
            pl.BlockSpec((_EB, D_EDGE), lambda i: (i, 0)),
            full((D_EDGE, D_EDGE)),
            full((1, D_GLOBAL)),
            full((D_GLOBAL, D_EDGE)),
            full((1, D_EDGE)),
        ],
        out_specs=pl.BlockSpec((_EB, D_EDGE), lambda i: (i, 0)),
        out_shape=jax.ShapeDtypeStruct((N_EDGES, D_EDGE), f32),
    )(edge_attr, gath, we_e, u, we_g, edge_b.reshape(1, D_EDGE))

    sp3 = sentp.reshape(NC, N_NODES, D_EDGE)
    rp3 = recvp.reshape(NC, N_NODES, D_EDGE)

    x_new, u_new = pl.pallas_call(
        _node_body,
        grid=(N_NODES // _NB,),
        in_specs=[
            pl.BlockSpec((_NB, D_FEAT), lambda i: (i, 0)),
            pl.BlockSpec((NC, _NB, D_EDGE), lambda i: (0, i, 0)),
            pl.BlockSpec((NC, _NB, D_EDGE), lambda i: (0, i, 0)),
            full((D_FEAT, D_FEAT)),
            full((D_EDGE, D_FEAT)),
            full((D_EDGE, D_FEAT)),
            full((D_FEAT, D_FEAT)),
            full((1, D_GLOBAL)),
            full((1, D_FEAT)),
            full((D_GLOBAL, D_GLOBAL)),
            full((D_FEAT, D_GLOBAL)),
            full((D_EDGE, D_GLOBAL)),
            full((1, D_GLOBAL)),
        ],
        out_specs=[
            pl.BlockSpec((_NB, D_FEAT), lambda i: (i, 0)),
            pl.BlockSpec((1, D_GLOBAL), lambda i: (0, 0)),
        ],
        out_shape=[
            jax.ShapeDtypeStruct((N_NODES, D_FEAT), f32),
            jax.ShapeDtypeStruct((1, D_GLOBAL), f32),
        ],
        scratch_shapes=[
            pltpu.VMEM((1, D_FEAT), f32),
            pltpu.VMEM((1, D_EDGE), f32),
        ],
    )(x, sp3, rp3, wn_x, wn_s, wn_r, wn_g, u, node_b.reshape(1, D_FEAT),
      wg_u, wg_x, wg_e, global_b.reshape(1, D_GLOBAL))

    return (x_new, edge_attr_new, u_new)
